# baseline (speedup 1.0000x reference)
import functools

import jax
import jax.numpy as jnp
from jax import lax
from jax.experimental import pallas as pl
from jax.experimental.pallas import tpu as pltpu
from jax.experimental.pallas import tpu_sc as plsc

N = 10000
E = 320000
H = 128
GROUP = 25
R_OUT = N // GROUP

NC = 2
NS = 16
NW = NC * NS
L = 16

CH = 128
NCHUNK = 80
NHALF = NCHUNK // 2
E_PAD = NW * NCHUNK * CH
N_ACC = NCHUNK * CH
ROWS_PER_TILE = N_ACC // NS

def _zero_rows(rows_ref):
    zeros16 = jnp.zeros((L,), jnp.float32)

    def zb(i, carry):
        rows_ref[i // (H // L), pl.ds((i % (H // L)) * L, L)] = zeros16
        return carry

    lax.fori_loop(0, CH * (H // L), zb, 0)


def _zero_acc_slice(rows_ref, acc, s):
    nfull = ROWS_PER_TILE // CH
    for k in range(nfull):
        pltpu.sync_copy(rows_ref, acc.at[pl.ds(s * ROWS_PER_TILE + k * CH, CH)])
    rem = ROWS_PER_TILE - nfull * CH
    if rem:
        pltpu.sync_copy(
            rows_ref.at[pl.ds(0, rem)],
            acc.at[pl.ds(s * ROWS_PER_TILE + nfull * CH, rem)],
        )


def _writeback(acc, out_hbm, c, s):
    pltpu.sync_copy(
        acc.at[pl.ds(s * ROWS_PER_TILE, ROWS_PER_TILE)],
        out_hbm.at[c, pl.ds(s * ROWS_PER_TILE, ROWS_PER_TILE)],
    )


def _edge_scatter_body(
    t_hbm, src_hbm, dst_hbm, out_hbm, src_v, dst_v, rows_a, acc, sem_a
):
    c = lax.axis_index("c")
    s = lax.axis_index("s")
    wid = s * NC + c

    _zero_rows(rows_a)
    _zero_acc_slice(rows_a, acc, s)
    pltpu.sync_copy(src_hbm.at[wid], src_v)
    pltpu.sync_copy(dst_hbm.at[wid], dst_v)
    plsc.subcore_barrier()

    def body(j, carry):
        pltpu.async_copy(t_hbm.at[src_v.at[j]], rows_a, sem_a).wait()
        pltpu.sync_copy(rows_a, acc.at[dst_v.at[j]], add=True)
        return carry

    lax.fori_loop(0, NCHUNK, body, 0)
    plsc.subcore_barrier()
    _writeback(acc, out_hbm, c, s)


@functools.cache
def _sc_kernels():
    mesh = plsc.VectorSubcoreMesh(
        core_axis_name="c", subcore_axis_name="s", num_cores=NC, num_subcores=NS
    )
    edge_scatter = pl.kernel(
        _edge_scatter_body,
        out_type=jax.ShapeDtypeStruct((NC, N_ACC, H), jnp.float32),
        mesh=mesh,
        scratch_types=[
            pltpu.VMEM((NCHUNK, CH), jnp.int32),
            pltpu.VMEM((NCHUNK, CH), jnp.int32),
            pltpu.VMEM((CH, H), jnp.float32),
            pltpu.VMEM_SHARED((N_ACC, H), jnp.float32),
            pltpu.SemaphoreType.DMA,
        ],
    )
    return edge_scatter


def _m1_body(x_ref, w_ref, d0_ref, d1_ref, u_ref, t_ref, di_ref, dg_ref):
    deg = d0_ref[...] + d1_ref[...] + 1.0
    dinv = 1.0 / jnp.sqrt(deg)
    u = jnp.dot(x_ref[...], w_ref[...], preferred_element_type=jnp.float32)
    u_ref[...] = u
    t_ref[...] = dinv * u
    di_ref[...] = dinv
    dg_ref[...] = 1.0 / deg


_m1 = pl.pallas_call(
    _m1_body,
    out_shape=[
        jax.ShapeDtypeStruct((N, H), jnp.float32),
        jax.ShapeDtypeStruct((N, H), jnp.float32),
        jax.ShapeDtypeStruct((N, 1), jnp.float32),
        jax.ShapeDtypeStruct((N, 1), jnp.float32),
    ],
)


def _m2_body(p0_ref, p1_ref, u1_ref, di_ref, dg_ref, b1_ref, w2_ref, u2_ref, t2_ref):
    di = di_ref[...]
    g = jnp.maximum(
        di * (p0_ref[...] + p1_ref[...]) + dg_ref[...] * u1_ref[...] + b1_ref[...],
        0.0,
    )
    u2 = jnp.dot(g, w2_ref[...], preferred_element_type=jnp.float32)
    u2_ref[...] = u2
    t2_ref[...] = di * u2


_m2 = pl.pallas_call(
    _m2_body,
    out_shape=[
        jax.ShapeDtypeStruct((N, H), jnp.float32),
        jax.ShapeDtypeStruct((N, H), jnp.float32),
    ],
)

_BLK = 200
_OBLK = _BLK // GROUP


def _m3_body(q0_ref, q1_ref, u2_ref, di_ref, dg_ref, b2_ref, wt_ref, bfc_ref, out_ref):
    g = jnp.maximum(
        di_ref[...] * (q0_ref[...] + q1_ref[...])
        + dg_ref[...] * u2_ref[...]
        + b2_ref[...],
        0.0,
    )
    p = g * wt_ref[...]
    row = lax.broadcasted_iota(jnp.int32, (_OBLK, _BLK), 0)
    col = lax.broadcasted_iota(jnp.int32, (_OBLK, _BLK), 1)
    sel = jnp.where(col // GROUP == row, 1.0, 0.0)
    out_ref[...] = (
        jnp.sum(jnp.dot(sel, p, preferred_element_type=jnp.float32), axis=1, keepdims=True)
        + bfc_ref[...]
    )


_m3 = pl.pallas_call(
    _m3_body,
    grid=(N // _BLK,),
    in_specs=[
        pl.BlockSpec((_BLK, H), lambda i: (i, 0)),
        pl.BlockSpec((_BLK, H), lambda i: (i, 0)),
        pl.BlockSpec((_BLK, H), lambda i: (i, 0)),
        pl.BlockSpec((_BLK, 1), lambda i: (i, 0)),
        pl.BlockSpec((_BLK, 1), lambda i: (i, 0)),
        pl.BlockSpec((1, H), lambda i: (0, 0)),
        pl.BlockSpec((_BLK, H), lambda i: (i, 0)),
        pl.BlockSpec((1, 1), lambda i: (0, 0)),
    ],
    out_specs=pl.BlockSpec((_OBLK, 1), lambda i: (i, 0)),
    out_shape=jax.ShapeDtypeStruct((R_OUT, 1), jnp.float32),
)


def kernel(x, edges, W1, b1, W2, b2, Wfc, bfc):
    pad = E_PAD - E
    src = jnp.concatenate(
        [edges[0], jnp.zeros((pad,), edges.dtype)]
    ).reshape(NW, NCHUNK, CH)
    dst = jnp.concatenate(
        [edges[1], jnp.full((pad,), N, edges.dtype)]
    ).reshape(NW, NCHUNK, CH)

    edge_scatter = _sc_kernels()
    degp = edge_scatter(jnp.ones((N, H), jnp.float32), dst, dst)
    d0 = degp[0, :N, :1]
    d1 = degp[1, :N, :1]

    u1, t1, dinv, dginv = _m1(x, W1, d0, d1)
    p = edge_scatter(t1, src, dst)
    u2, t2 = _m2(p[0, :N], p[1, :N], u1, dinv, dginv, b1.reshape(1, H), W2)
    q = edge_scatter(t2, src, dst)
    wt = jnp.tile(Wfc.reshape(GROUP, H), (R_OUT, 1))
    out = _m3(
        q[0, :N], q[1, :N], u2, dinv, dginv, b2.reshape(1, H), wt, bfc.reshape(1, 1)
    )
    return out


# speedup vs baseline: 2.6195x; 2.6195x over previous
import functools

import jax
import jax.numpy as jnp
from jax import lax
from jax.experimental import pallas as pl
from jax.experimental.pallas import tpu as pltpu
from jax.experimental.pallas import tpu_sc as plsc

N = 10000
E = 320000
H = 128
GROUP = 25
R_OUT = N // GROUP

NC = 2
NS = 16
NW = NC * NS
L = 16

CH = 128
NCHUNK = 80
NHALF = NCHUNK // 2
E_PAD = NW * NCHUNK * CH
N_ACC = NCHUNK * CH
ROWS_PER_TILE = N_ACC // NS

def _zero_rows(rows_ref):
    zeros16 = jnp.zeros((L,), jnp.float32)

    def zb(i, carry):
        rows_ref[i // (H // L), pl.ds((i % (H // L)) * L, L)] = zeros16
        return carry

    lax.fori_loop(0, CH * (H // L), zb, 0)


def _zero_acc_slice(rows_ref, acc, s):
    nfull = ROWS_PER_TILE // CH
    for k in range(nfull):
        pltpu.sync_copy(rows_ref, acc.at[pl.ds(s * ROWS_PER_TILE + k * CH, CH)])
    rem = ROWS_PER_TILE - nfull * CH
    if rem:
        pltpu.sync_copy(
            rows_ref.at[pl.ds(0, rem)],
            acc.at[pl.ds(s * ROWS_PER_TILE + nfull * CH, rem)],
        )


def _writeback(acc, out_hbm, c, s):
    pltpu.sync_copy(
        acc.at[pl.ds(s * ROWS_PER_TILE, ROWS_PER_TILE)],
        out_hbm.at[c, pl.ds(s * ROWS_PER_TILE, ROWS_PER_TILE)],
    )


def _edge_scatter_body(
    t_hbm, src_hbm, dst_hbm, out_hbm, src_v, dst_v, rows_a, acc, sem_a
):
    c = lax.axis_index("c")
    s = lax.axis_index("s")
    wid = s * NC + c

    _zero_rows(rows_a)
    _zero_acc_slice(rows_a, acc, s)
    pltpu.sync_copy(src_hbm.at[wid], src_v)
    pltpu.sync_copy(dst_hbm.at[wid], dst_v)
    plsc.subcore_barrier()

    def body(j, carry):
        pltpu.async_copy(t_hbm.at[src_v.at[j]], rows_a, sem_a).wait()
        pltpu.sync_copy(rows_a, acc.at[dst_v.at[j]], add=True)
        return carry

    lax.fori_loop(0, NCHUNK, body, 0)
    plsc.subcore_barrier()
    _writeback(acc, out_hbm, c, s)


@functools.cache
def _sc_kernels():
    mesh = plsc.VectorSubcoreMesh(
        core_axis_name="c", subcore_axis_name="s", num_cores=NC, num_subcores=NS
    )
    edge_scatter = pl.kernel(
        _edge_scatter_body,
        out_type=jax.ShapeDtypeStruct((NC, N_ACC, H), jnp.float32),
        mesh=mesh,
        scratch_types=[
            pltpu.VMEM((NCHUNK, CH), jnp.int32),
            pltpu.VMEM((NCHUNK, CH), jnp.int32),
            pltpu.VMEM((CH, H), jnp.float32),
            pltpu.VMEM_SHARED((N_ACC, H), jnp.float32),
            pltpu.SemaphoreType.DMA,
        ],
    )
    return edge_scatter


def _m1_body(x_ref, w_ref, d0_ref, d1_ref, u_ref, t_ref, di_ref, dg_ref):
    deg = d0_ref[...] + d1_ref[...] + 1.0
    dinv = 1.0 / jnp.sqrt(deg)
    u = jnp.dot(x_ref[...], w_ref[...], preferred_element_type=jnp.float32)
    u_ref[...] = u
    t_ref[...] = dinv * u
    di_ref[...] = dinv
    dg_ref[...] = 1.0 / deg


_m1 = pl.pallas_call(
    _m1_body,
    out_shape=[
        jax.ShapeDtypeStruct((N, H), jnp.float32),
        jax.ShapeDtypeStruct((N, H), jnp.float32),
        jax.ShapeDtypeStruct((N, 1), jnp.float32),
        jax.ShapeDtypeStruct((N, 1), jnp.float32),
    ],
)


def _m2_body(p0_ref, p1_ref, u1_ref, di_ref, dg_ref, b1_ref, w2_ref, u2_ref, t2_ref):
    di = di_ref[...]
    g = jnp.maximum(
        di * (p0_ref[...] + p1_ref[...]) + dg_ref[...] * u1_ref[...] + b1_ref[...],
        0.0,
    )
    u2 = jnp.dot(g, w2_ref[...], preferred_element_type=jnp.float32)
    u2_ref[...] = u2
    t2_ref[...] = di * u2


_m2 = pl.pallas_call(
    _m2_body,
    out_shape=[
        jax.ShapeDtypeStruct((N, H), jnp.float32),
        jax.ShapeDtypeStruct((N, H), jnp.float32),
    ],
)

def _m3_body(q0_ref, q1_ref, u2_ref, di_ref, dg_ref, b2_ref, g2_ref):
    g2_ref[...] = jnp.maximum(
        di_ref[...] * (q0_ref[...] + q1_ref[...])
        + dg_ref[...] * u2_ref[...]
        + b2_ref[...],
        0.0,
    )


_m3 = pl.pallas_call(
    _m3_body,
    out_shape=jax.ShapeDtypeStruct((N, H), jnp.float32),
)


def _m4_body(g_ref, w_ref, bfc_ref, out_ref):
    out_ref[...] = jnp.dot(g_ref[...], w_ref[...]) + bfc_ref[...]


_m4 = pl.pallas_call(
    _m4_body,
    out_shape=jax.ShapeDtypeStruct((R_OUT, 1), jnp.float32),
)


def kernel(x, edges, W1, b1, W2, b2, Wfc, bfc):
    pad = E_PAD - E
    pad_src = jnp.arange(pad, dtype=edges.dtype) % N
    pad_dst = N + jnp.arange(pad, dtype=edges.dtype) % (N_ACC - N)
    src = jnp.concatenate([edges[0], pad_src]).reshape(NW, NCHUNK, CH)
    dst = jnp.concatenate([edges[1], pad_dst]).reshape(NW, NCHUNK, CH)

    edge_scatter = _sc_kernels()
    degp = edge_scatter(jnp.ones((N, H), jnp.float32), dst, dst)
    d0 = degp[0, :N, :1]
    d1 = degp[1, :N, :1]

    u1, t1, dinv, dginv = _m1(x, W1, d0, d1)
    p = edge_scatter(t1, src, dst)
    u2, t2 = _m2(p[0, :N], p[1, :N], u1, dinv, dginv, b1.reshape(1, H), W2)
    q = edge_scatter(t2, src, dst)
    g2 = _m3(q[0, :N], q[1, :N], u2, dinv, dginv, b2.reshape(1, H))
    out = _m4(g2.reshape(R_OUT, H * GROUP), Wfc, bfc.reshape(1, 1))
    return out


# speedup vs baseline: 3.7919x; 1.4476x over previous
import functools

import jax
import jax.numpy as jnp
from jax import lax
from jax.experimental import pallas as pl
from jax.experimental.pallas import tpu as pltpu
from jax.experimental.pallas import tpu_sc as plsc

N = 10000
E = 320000
H = 128
GROUP = 25
R_OUT = N // GROUP

NC = 2
NS = 16
NW = NC * NS
L = 16

CH = 128
NCHUNK = 80
NHALF = NCHUNK // 2
E_PAD = NW * NCHUNK * CH
N_ACC = NCHUNK * CH
ROWS_PER_TILE = N_ACC // NS

def _zero_rows(rows_ref):
    zeros16 = jnp.zeros((L,), jnp.float32)

    def zb(i, carry):
        rows_ref[i // (H // L), pl.ds((i % (H // L)) * L, L)] = zeros16
        return carry

    lax.fori_loop(0, CH * (H // L), zb, 0)


def _zero_acc_slice(rows_ref, acc, s):
    nfull = ROWS_PER_TILE // CH
    for k in range(nfull):
        pltpu.sync_copy(rows_ref, acc.at[pl.ds(s * ROWS_PER_TILE + k * CH, CH)])
    rem = ROWS_PER_TILE - nfull * CH
    if rem:
        pltpu.sync_copy(
            rows_ref.at[pl.ds(0, rem)],
            acc.at[pl.ds(s * ROWS_PER_TILE + nfull * CH, rem)],
        )


def _writeback(acc, out_hbm, c, s):
    pltpu.sync_copy(
        acc.at[pl.ds(s * ROWS_PER_TILE, ROWS_PER_TILE)],
        out_hbm.at[c, pl.ds(s * ROWS_PER_TILE, ROWS_PER_TILE)],
    )


def _edge_scatter_body(
    t_hbm, src_hbm, dst_hbm, out_hbm, src_v, dst_v, rows_a, rows_b, acc, sem_a, sem_b
):
    c = lax.axis_index("c")
    s = lax.axis_index("s")
    wid = s * NC + c

    _zero_rows(rows_a)
    _zero_acc_slice(rows_a, acc, s)
    plsc.subcore_barrier()

    for half in range(2):
        pltpu.sync_copy(src_hbm.at[wid, pl.ds(half * NHALF, NHALF)], src_v)
        pltpu.sync_copy(dst_hbm.at[wid, pl.ds(half * NHALF, NHALF)], dst_v)
        pltpu.async_copy(t_hbm.at[src_v.at[0]], rows_a, sem_a)

        def body(i, carry):
            k = 2 * i
            pltpu.async_copy(t_hbm.at[src_v.at[k + 1]], rows_b, sem_b)
            pltpu.make_async_copy(t_hbm.at[src_v.at[k]], rows_a, sem_a).wait()
            pltpu.sync_copy(rows_a, acc.at[dst_v.at[k]], add=True)
            pltpu.async_copy(t_hbm.at[src_v.at[k + 2]], rows_a, sem_a)
            pltpu.make_async_copy(t_hbm.at[src_v.at[k + 1]], rows_b, sem_b).wait()
            pltpu.sync_copy(rows_b, acc.at[dst_v.at[k + 1]], add=True)
            return carry

        lax.fori_loop(0, NHALF // 2 - 1, body, 0)
        pltpu.async_copy(t_hbm.at[src_v.at[NHALF - 1]], rows_b, sem_b)
        pltpu.make_async_copy(t_hbm.at[src_v.at[NHALF - 2]], rows_a, sem_a).wait()
        pltpu.sync_copy(rows_a, acc.at[dst_v.at[NHALF - 2]], add=True)
        pltpu.make_async_copy(t_hbm.at[src_v.at[NHALF - 1]], rows_b, sem_b).wait()
        pltpu.sync_copy(rows_b, acc.at[dst_v.at[NHALF - 1]], add=True)
    plsc.subcore_barrier()
    _writeback(acc, out_hbm, c, s)


@functools.cache
def _sc_kernels():
    mesh = plsc.VectorSubcoreMesh(
        core_axis_name="c", subcore_axis_name="s", num_cores=NC, num_subcores=NS
    )
    edge_scatter = pl.kernel(
        _edge_scatter_body,
        out_type=jax.ShapeDtypeStruct((NC, N_ACC, H), jnp.float32),
        mesh=mesh,
        scratch_types=[
            pltpu.VMEM((NHALF, CH), jnp.int32),
            pltpu.VMEM((NHALF, CH), jnp.int32),
            pltpu.VMEM((CH, H), jnp.float32),
            pltpu.VMEM((CH, H), jnp.float32),
            pltpu.VMEM_SHARED((N_ACC, H), jnp.float32),
            pltpu.SemaphoreType.DMA,
            pltpu.SemaphoreType.DMA,
        ],
    )
    return edge_scatter


def _m1_body(x_ref, w_ref, d0_ref, d1_ref, u_ref, t_ref, di_ref, dg_ref):
    deg = d0_ref[...] + d1_ref[...] + 1.0
    dinv = 1.0 / jnp.sqrt(deg)
    u = jnp.dot(x_ref[...], w_ref[...], preferred_element_type=jnp.float32)
    u_ref[...] = u
    t_ref[...] = dinv * u
    di_ref[...] = dinv
    dg_ref[...] = 1.0 / deg


_m1 = pl.pallas_call(
    _m1_body,
    out_shape=[
        jax.ShapeDtypeStruct((N, H), jnp.float32),
        jax.ShapeDtypeStruct((N, H), jnp.float32),
        jax.ShapeDtypeStruct((N, 1), jnp.float32),
        jax.ShapeDtypeStruct((N, 1), jnp.float32),
    ],
)


def _m2_body(p0_ref, p1_ref, u1_ref, di_ref, dg_ref, b1_ref, w2_ref, u2_ref, t2_ref):
    di = di_ref[...]
    g = jnp.maximum(
        di * (p0_ref[...] + p1_ref[...]) + dg_ref[...] * u1_ref[...] + b1_ref[...],
        0.0,
    )
    u2 = jnp.dot(g, w2_ref[...], preferred_element_type=jnp.float32)
    u2_ref[...] = u2
    t2_ref[...] = di * u2


_m2 = pl.pallas_call(
    _m2_body,
    out_shape=[
        jax.ShapeDtypeStruct((N, H), jnp.float32),
        jax.ShapeDtypeStruct((N, H), jnp.float32),
    ],
)

def _m3_body(q0_ref, q1_ref, u2_ref, di_ref, dg_ref, b2_ref, g2_ref):
    g2_ref[...] = jnp.maximum(
        di_ref[...] * (q0_ref[...] + q1_ref[...])
        + dg_ref[...] * u2_ref[...]
        + b2_ref[...],
        0.0,
    )


_m3 = pl.pallas_call(
    _m3_body,
    out_shape=jax.ShapeDtypeStruct((N, H), jnp.float32),
)


def _m4_body(g_ref, w_ref, bfc_ref, out_ref):
    out_ref[...] = jnp.dot(g_ref[...], w_ref[...]) + bfc_ref[...]


_m4 = pl.pallas_call(
    _m4_body,
    out_shape=jax.ShapeDtypeStruct((R_OUT, 1), jnp.float32),
)


def kernel(x, edges, W1, b1, W2, b2, Wfc, bfc):
    pad = E_PAD - E
    pad_src = jnp.arange(pad, dtype=edges.dtype) % N
    pad_dst = N + jnp.arange(pad, dtype=edges.dtype) % (N_ACC - N)
    src = jnp.concatenate([edges[0], pad_src]).reshape(NW, NCHUNK, CH)
    dst = jnp.concatenate([edges[1], pad_dst]).reshape(NW, NCHUNK, CH)

    edge_scatter = _sc_kernels()
    degp = edge_scatter(jnp.ones((N, H), jnp.float32), dst, dst)
    d0 = degp[0, :N, :1]
    d1 = degp[1, :N, :1]

    u1, t1, dinv, dginv = _m1(x, W1, d0, d1)
    p = edge_scatter(t1, src, dst)
    u2, t2 = _m2(p[0, :N], p[1, :N], u1, dinv, dginv, b1.reshape(1, H), W2)
    q = edge_scatter(t2, src, dst)
    g2 = _m3(q[0, :N], q[1, :N], u2, dinv, dginv, b2.reshape(1, H))
    out = _m4(g2.reshape(R_OUT, H * GROUP), Wfc, bfc.reshape(1, 1))
    return out


# speedup vs baseline: 4.1753x; 1.1011x over previous
import functools

import jax
import jax.numpy as jnp
from jax import lax
from jax.experimental import pallas as pl
from jax.experimental.pallas import tpu as pltpu
from jax.experimental.pallas import tpu_sc as plsc

N = 10000
E = 320000
H = 128
GROUP = 25
R_OUT = N // GROUP

NC = 2
NS = 16
NW = NC * NS
L = 16

CH = 128
NCHUNK = 80
NHALF = NCHUNK // 2
E_PAD = NW * NCHUNK * CH
N_ACC = NCHUNK * CH
ROWS_PER_TILE = N_ACC // NS

def _zero_rows(rows_ref):
    zeros16 = jnp.zeros((L,), jnp.float32)

    def zb(i, carry):
        rows_ref[i // (H // L), pl.ds((i % (H // L)) * L, L)] = zeros16
        return carry

    lax.fori_loop(0, CH * (H // L), zb, 0)


def _zero_acc_slice(rows_ref, acc, s):
    nfull = ROWS_PER_TILE // CH
    for k in range(nfull):
        pltpu.sync_copy(rows_ref, acc.at[pl.ds(s * ROWS_PER_TILE + k * CH, CH)])
    rem = ROWS_PER_TILE - nfull * CH
    if rem:
        pltpu.sync_copy(
            rows_ref.at[pl.ds(0, rem)],
            acc.at[pl.ds(s * ROWS_PER_TILE + nfull * CH, rem)],
        )


def _writeback(acc, out_hbm, c, s):
    pltpu.sync_copy(
        acc.at[pl.ds(s * ROWS_PER_TILE, ROWS_PER_TILE)],
        out_hbm.at[c, pl.ds(s * ROWS_PER_TILE, ROWS_PER_TILE)],
    )


def _edge_scatter_body(
    t_hbm, src_hbm, dst_hbm, out_hbm, src_v, dst_v, rows_a, rows_b, acc, sem_a, sem_b
):
    c = lax.axis_index("c")
    s = lax.axis_index("s")
    wid = s * NC + c

    _zero_rows(rows_a)
    _zero_acc_slice(rows_a, acc, s)
    plsc.subcore_barrier()

    for half in range(2):
        pltpu.sync_copy(src_hbm.at[wid, pl.ds(half * NHALF, NHALF)], src_v)
        pltpu.sync_copy(dst_hbm.at[wid, pl.ds(half * NHALF, NHALF)], dst_v)
        pltpu.async_copy(t_hbm.at[src_v.at[0]], rows_a, sem_a)

        def body(i, carry):
            k = 2 * i
            pltpu.async_copy(t_hbm.at[src_v.at[k + 1]], rows_b, sem_b)
            pltpu.make_async_copy(t_hbm.at[src_v.at[k]], rows_a, sem_a).wait()
            pltpu.sync_copy(rows_a, acc.at[dst_v.at[k]], add=True)
            pltpu.async_copy(t_hbm.at[src_v.at[k + 2]], rows_a, sem_a)
            pltpu.make_async_copy(t_hbm.at[src_v.at[k + 1]], rows_b, sem_b).wait()
            pltpu.sync_copy(rows_b, acc.at[dst_v.at[k + 1]], add=True)
            return carry

        lax.fori_loop(0, NHALF // 2 - 1, body, 0)
        pltpu.async_copy(t_hbm.at[src_v.at[NHALF - 1]], rows_b, sem_b)
        pltpu.make_async_copy(t_hbm.at[src_v.at[NHALF - 2]], rows_a, sem_a).wait()
        pltpu.sync_copy(rows_a, acc.at[dst_v.at[NHALF - 2]], add=True)
        pltpu.make_async_copy(t_hbm.at[src_v.at[NHALF - 1]], rows_b, sem_b).wait()
        pltpu.sync_copy(rows_b, acc.at[dst_v.at[NHALF - 1]], add=True)
    plsc.subcore_barrier()
    _writeback(acc, out_hbm, c, s)


def _deg_scatter_body(dst_hbm, out_hbm, dst_v, rows_v, acc):
    c = lax.axis_index("c")
    s = lax.axis_index("s")
    wid = s * NC + c

    _zero_rows(rows_v)
    _zero_acc_slice(rows_v, acc, s)
    pltpu.sync_copy(dst_hbm.at[wid], dst_v)
    ones16 = jnp.ones((L,), jnp.float32)

    def ob(i, carry):
        rows_v[i // (H // L), pl.ds((i % (H // L)) * L, L)] = ones16
        return carry

    lax.fori_loop(0, CH * (H // L), ob, 0)
    plsc.subcore_barrier()

    def body(j, carry):
        pltpu.sync_copy(rows_v, acc.at[dst_v.at[j]], add=True)
        return carry

    lax.fori_loop(0, NCHUNK, body, 0)
    plsc.subcore_barrier()
    _writeback(acc, out_hbm, c, s)


@functools.cache
def _sc_kernels():
    mesh = plsc.VectorSubcoreMesh(
        core_axis_name="c", subcore_axis_name="s", num_cores=NC, num_subcores=NS
    )
    edge_scatter = pl.kernel(
        _edge_scatter_body,
        out_type=jax.ShapeDtypeStruct((NC, N_ACC, H), jnp.float32),
        mesh=mesh,
        scratch_types=[
            pltpu.VMEM((NHALF, CH), jnp.int32),
            pltpu.VMEM((NHALF, CH), jnp.int32),
            pltpu.VMEM((CH, H), jnp.float32),
            pltpu.VMEM((CH, H), jnp.float32),
            pltpu.VMEM_SHARED((N_ACC, H), jnp.float32),
            pltpu.SemaphoreType.DMA,
            pltpu.SemaphoreType.DMA,
        ],
    )
    deg_scatter = pl.kernel(
        _deg_scatter_body,
        out_type=jax.ShapeDtypeStruct((NC, N_ACC, H), jnp.float32),
        mesh=mesh,
        scratch_types=[
            pltpu.VMEM((NCHUNK, CH), jnp.int32),
            pltpu.VMEM((CH, H), jnp.float32),
            pltpu.VMEM_SHARED((N_ACC, H), jnp.float32),
        ],
    )
    return edge_scatter, deg_scatter


def _m1_body(x_ref, w_ref, d0_ref, d1_ref, u_ref, t_ref, di_ref, dg_ref):
    deg = d0_ref[...] + d1_ref[...] + 1.0
    dinv = 1.0 / jnp.sqrt(deg)
    u = jnp.dot(x_ref[...], w_ref[...], preferred_element_type=jnp.float32)
    u_ref[...] = u
    t_ref[...] = dinv * u
    di_ref[...] = dinv
    dg_ref[...] = 1.0 / deg


_m1 = pl.pallas_call(
    _m1_body,
    out_shape=[
        jax.ShapeDtypeStruct((N, H), jnp.float32),
        jax.ShapeDtypeStruct((N, H), jnp.float32),
        jax.ShapeDtypeStruct((N, 1), jnp.float32),
        jax.ShapeDtypeStruct((N, 1), jnp.float32),
    ],
)


def _m2_body(p0_ref, p1_ref, u1_ref, di_ref, dg_ref, b1_ref, w2_ref, u2_ref, t2_ref):
    di = di_ref[...]
    g = jnp.maximum(
        di * (p0_ref[...] + p1_ref[...]) + dg_ref[...] * u1_ref[...] + b1_ref[...],
        0.0,
    )
    u2 = jnp.dot(g, w2_ref[...], preferred_element_type=jnp.float32)
    u2_ref[...] = u2
    t2_ref[...] = di * u2


_m2 = pl.pallas_call(
    _m2_body,
    out_shape=[
        jax.ShapeDtypeStruct((N, H), jnp.float32),
        jax.ShapeDtypeStruct((N, H), jnp.float32),
    ],
)

def _m3_body(q0_ref, q1_ref, u2_ref, di_ref, dg_ref, b2_ref, g2_ref):
    g2_ref[...] = jnp.maximum(
        di_ref[...] * (q0_ref[...] + q1_ref[...])
        + dg_ref[...] * u2_ref[...]
        + b2_ref[...],
        0.0,
    )


_m3 = pl.pallas_call(
    _m3_body,
    out_shape=jax.ShapeDtypeStruct((N, H), jnp.float32),
)


def _m4_body(g_ref, w_ref, bfc_ref, out_ref):
    out_ref[...] = jnp.dot(g_ref[...], w_ref[...]) + bfc_ref[...]


_m4 = pl.pallas_call(
    _m4_body,
    out_shape=jax.ShapeDtypeStruct((R_OUT, 1), jnp.float32),
)


def kernel(x, edges, W1, b1, W2, b2, Wfc, bfc):
    pad = E_PAD - E
    pad_src = jnp.arange(pad, dtype=edges.dtype) % N
    pad_dst = N + jnp.arange(pad, dtype=edges.dtype) % (N_ACC - N)
    src = jnp.concatenate([edges[0], pad_src]).reshape(NW, NCHUNK, CH)
    dst = jnp.concatenate([edges[1], pad_dst]).reshape(NW, NCHUNK, CH)

    edge_scatter, deg_scatter = _sc_kernels()
    degp = deg_scatter(dst)
    d0 = degp[0, :N, :1]
    d1 = degp[1, :N, :1]

    u1, t1, dinv, dginv = _m1(x, W1, d0, d1)
    p = edge_scatter(t1, src, dst)
    u2, t2 = _m2(p[0, :N], p[1, :N], u1, dinv, dginv, b1.reshape(1, H), W2)
    q = edge_scatter(t2, src, dst)
    g2 = _m3(q[0, :N], q[1, :N], u2, dinv, dginv, b2.reshape(1, H))
    out = _m4(g2.reshape(R_OUT, H * GROUP), Wfc, bfc.reshape(1, 1))
    return out
